# baseline (speedup 1.0000x reference)
import functools

import jax
import jax.numpy as jnp
from jax.experimental import pallas as pl
from jax.experimental.pallas import tpu as pltpu

_EDGE_TILE = 25600
_NODE_CHUNK = 800
_NB = 136
_REP = 8
_STRIDE = 72
_NBUF = 8
_AHEAD = 6


def _copy_in(src_hbm, attr_hbm, src_buf, attr_buf, ssem, asem, tile, slot):
    cs = pltpu.make_async_copy(src_hbm.at[tile], src_buf.at[slot],
                               ssem.at[slot])
    ca = pltpu.make_async_copy(attr_hbm.at[tile], attr_buf.at[slot],
                               asem.at[slot])
    return cs, ca


def _fused_kernel(num_tiles, num_graphs, node_rows,
                  src_hbm, attr_hbm, batch_ref, w1_ref, b1_ref, w2_ref,
                  b2_ref, w3_ref, b3_ref, out_ref,
                  starts_ref, acc_ref, src_buf, attr_buf, ssem, asem):
    i = pl.program_id(0)
    hidden = w2_ref.shape[0]

    @pl.when(i == 0)
    def _init():
        gcol = jax.lax.broadcasted_iota(jnp.int32, (_NB, 1), 0)

        def body(j, s):
            row = batch_ref[pl.ds(j, 1), :]
            return s + jnp.sum((gcol > row).astype(jnp.int32), axis=1,
                               keepdims=True)

        zero = jnp.zeros((_NB, 1), jnp.int32)
        starts_ref[...] = jax.lax.fori_loop(0, node_rows, body, zero)
        acc_ref[...] = jnp.zeros_like(acc_ref)

        def prologue(t, _):
            cs, ca = _copy_in(src_hbm, attr_hbm, src_buf, attr_buf,
                              ssem, asem, t, t % _NBUF)
            cs.start()
            ca.start()
            return 0

        jax.lax.fori_loop(0, min(_AHEAD, num_tiles), prologue, 0)

    nxt = i + _AHEAD

    @pl.when(nxt < num_tiles)
    def _prefetch():
        cs, ca = _copy_in(src_hbm, attr_hbm, src_buf, attr_buf,
                          ssem, asem, nxt, nxt % _NBUF)
        cs.start()
        ca.start()

    slot = i % _NBUF
    cs, ca = _copy_in(src_hbm, attr_hbm, src_buf, attr_buf,
                      ssem, asem, i, slot)
    cs.wait()
    ca.wait()

    spt = jnp.transpose(src_buf[slot], (1, 0))
    ap = attr_buf[slot]
    hp = jnp.dot(ap, w1_ref[...], preferred_element_type=jnp.float32)
    hp = jnp.maximum(hp + b1_ref[...], 0.0)
    hp16 = hp.astype(jnp.bfloat16)
    one = jnp.ones((), jnp.float32)
    zro = jnp.zeros((), jnp.float32)
    starts = starts_ref[...]
    total = None
    for k in range(_REP):
        u_k = jnp.where(spt[k:k + 1, :] >= starts, one, zro)
        h_k = hp16[:, _STRIDE * k:_STRIDE * k + hidden + 1]
        p_k = jax.lax.dot_general(u_k.astype(jnp.bfloat16), h_k,
                                  (((1,), (0,)), ((), ())),
                                  preferred_element_type=jnp.float32)
        total = p_k if total is None else total + p_k
    acc_ref[...] += total[0:num_graphs, :] - total[1:num_graphs + 1, :]

    @pl.when(i == num_tiles - 1)
    def _finish():
        a = acc_ref[...]
        gf = a[:, 0:hidden] / jnp.maximum(a[:, hidden:hidden + 1], 1.0)
        h2 = jnp.dot(gf, w2_ref[...], preferred_element_type=jnp.float32)
        h2 = jnp.maximum(h2 + b2_ref[...], 0.0)
        out = jnp.dot(h2, w3_ref[...], preferred_element_type=jnp.float32)
        out_ref[...] = out + b3_ref[...]


@jax.jit
def kernel(edge_index, edge_attr, batch, W1, b1, W2, b2, W3, b3):
    n_edges, in_dim = edge_attr.shape
    n_nodes = batch.shape[0]
    hidden = W1.shape[1]
    out_dim = W3.shape[1]
    num_graphs = 128
    rep = _REP
    assert 128 // in_dim == rep

    et = _EDGE_TILE
    pad_e = (-n_edges) % et
    src = edge_index[0].astype(jnp.int32)
    attr = edge_attr.astype(jnp.bfloat16)
    if pad_e:
        src = jnp.concatenate(
            [src, jnp.full((pad_e,), n_nodes, jnp.int32)])
        attr = jnp.concatenate(
            [attr, jnp.zeros((pad_e, in_dim), attr.dtype)])
    num_tiles = (n_edges + pad_e) // et
    epr = et // rep
    src = src.reshape(num_tiles, epr, rep)
    attr = attr.reshape(num_tiles, epr, 128)

    w1b = jnp.zeros((128, rep * _STRIDE), jnp.float32)
    b1a = jnp.zeros((1, rep * _STRIDE), jnp.float32)
    for k in range(rep):
        w1b = w1b.at[in_dim * k:in_dim * (k + 1),
                     _STRIDE * k:_STRIDE * k + hidden].set(W1)
        b1a = b1a.at[0, _STRIDE * k:_STRIDE * k + hidden].set(b1)
        b1a = b1a.at[0, _STRIDE * k + hidden].set(1.0)
    w1b = w1b.astype(jnp.bfloat16)

    b32 = batch.astype(jnp.int32)
    pad_n = (-n_nodes) % _NODE_CHUNK
    if pad_n:
        b32 = jnp.concatenate(
            [b32, jnp.full((pad_n,), num_graphs, jnp.int32)])
    node_rows = (n_nodes + pad_n) // _NODE_CHUNK
    b32 = b32.reshape(node_rows, _NODE_CHUNK)

    const = lambda i: (0, 0)
    out = pl.pallas_call(
        functools.partial(_fused_kernel, num_tiles, num_graphs, node_rows),
        grid=(num_tiles,),
        in_specs=[
            pl.BlockSpec(memory_space=pl.ANY),
            pl.BlockSpec(memory_space=pl.ANY),
            pl.BlockSpec((node_rows, _NODE_CHUNK), const),
            pl.BlockSpec((128, rep * _STRIDE), const),
            pl.BlockSpec((1, rep * _STRIDE), const),
            pl.BlockSpec((hidden, hidden), const),
            pl.BlockSpec((1, hidden), const),
            pl.BlockSpec((hidden, out_dim), const),
            pl.BlockSpec((1, out_dim), const),
        ],
        out_specs=pl.BlockSpec((num_graphs, out_dim), const),
        out_shape=jax.ShapeDtypeStruct((num_graphs, out_dim), jnp.float32),
        scratch_shapes=[
            pltpu.VMEM((_NB, 1), jnp.int32),
            pltpu.VMEM((num_graphs, hidden + 1), jnp.float32),
            pltpu.VMEM((_NBUF, epr, rep), jnp.int32),
            pltpu.VMEM((_NBUF, epr, 128), jnp.bfloat16),
            pltpu.SemaphoreType.DMA((_NBUF,)),
            pltpu.SemaphoreType.DMA((_NBUF,)),
        ],
        compiler_params=pltpu.CompilerParams(
            dimension_semantics=("arbitrary",)),
    )(src, attr, b32, w1b, b1a, W2,
      b2.reshape(1, hidden), W3, b3.reshape(1, out_dim))
    return out


# speedup vs baseline: 1.4570x; 1.4570x over previous
import functools

import jax
import jax.numpy as jnp
from jax.experimental import pallas as pl
from jax.experimental.pallas import tpu as pltpu

_EDGE_TILE = 25600
_NODE_CHUNK = 800
_NB = 136


def _fused_kernel(num_tiles, num_graphs, node_rows,
                  src_ref, attr_ref, batch_ref, w1_ref, b1_ref, w2_ref,
                  b2_ref, w3_ref, b3_ref, out_ref, starts_ref, acc_ref):
    i = pl.program_id(0)
    et = attr_ref.shape[0]
    hidden = w2_ref.shape[0]

    @pl.when(i == 0)
    def _init():
        gcol = jax.lax.broadcasted_iota(jnp.int32, (_NB, 1), 0)

        def body(j, s):
            row = batch_ref[pl.ds(j, 1), :]
            return s + jnp.sum((gcol > row).astype(jnp.int32), axis=1,
                               keepdims=True)

        zero = jnp.zeros((_NB, 1), jnp.int32)
        starts_ref[...] = jax.lax.fori_loop(0, node_rows, body, zero)
        acc_ref[...] = jnp.zeros_like(acc_ref)

    node = src_ref[0]
    one = jnp.ones((), jnp.float32)
    zro = jnp.zeros((), jnp.float32)
    u = jnp.where(node >= starts_ref[...], one, zro).astype(jnp.bfloat16)
    h = jnp.dot(attr_ref[...], w1_ref[...],
                preferred_element_type=jnp.float32)
    h = jnp.maximum(h + b1_ref[...], 0.0)
    h_aug = jnp.concatenate(
        [h.astype(jnp.bfloat16), jnp.ones((et, 1), jnp.bfloat16)], axis=1)
    p = jax.lax.dot_general(u, h_aug, (((1,), (0,)), ((), ())),
                            preferred_element_type=jnp.float32)
    acc_ref[...] += p[0:num_graphs, :] - p[1:num_graphs + 1, :]

    @pl.when(i == num_tiles - 1)
    def _finish():
        a = acc_ref[...]
        gf = a[:, 0:hidden] / jnp.maximum(a[:, hidden:hidden + 1], 1.0)
        h2 = jnp.dot(gf, w2_ref[...], preferred_element_type=jnp.float32)
        h2 = jnp.maximum(h2 + b2_ref[...], 0.0)
        out = jnp.dot(h2, w3_ref[...], preferred_element_type=jnp.float32)
        out_ref[...] = out + b3_ref[...]


@jax.jit
def kernel(edge_index, edge_attr, batch, W1, b1, W2, b2, W3, b3):
    n_edges, in_dim = edge_attr.shape
    n_nodes = batch.shape[0]
    hidden = W1.shape[1]
    out_dim = W3.shape[1]
    num_graphs = 128

    et = _EDGE_TILE
    pad_e = (-n_edges) % et
    src = edge_index[0].astype(jnp.int32)
    attr = edge_attr.astype(jnp.bfloat16)
    if pad_e:
        src = jnp.concatenate(
            [src, jnp.full((pad_e,), n_nodes, jnp.int32)])
        attr = jnp.concatenate(
            [attr, jnp.zeros((pad_e, in_dim), attr.dtype)])
    num_tiles = (n_edges + pad_e) // et
    src = src.reshape(num_tiles, 1, et)

    b32 = batch.astype(jnp.int32)
    pad_n = (-n_nodes) % _NODE_CHUNK
    if pad_n:
        b32 = jnp.concatenate(
            [b32, jnp.full((pad_n,), num_graphs, jnp.int32)])
    node_rows = (n_nodes + pad_n) // _NODE_CHUNK
    b32 = b32.reshape(node_rows, _NODE_CHUNK)

    const = lambda i: (0, 0)
    out = pl.pallas_call(
        functools.partial(_fused_kernel, num_tiles, num_graphs, node_rows),
        grid=(num_tiles,),
        in_specs=[
            pl.BlockSpec((1, 1, et), lambda i: (i, 0, 0)),
            pl.BlockSpec((et, in_dim), lambda i: (i, 0)),
            pl.BlockSpec((node_rows, _NODE_CHUNK), const),
            pl.BlockSpec((in_dim, hidden), const),
            pl.BlockSpec((1, hidden), const),
            pl.BlockSpec((hidden, hidden), const),
            pl.BlockSpec((1, hidden), const),
            pl.BlockSpec((hidden, out_dim), const),
            pl.BlockSpec((1, out_dim), const),
        ],
        out_specs=pl.BlockSpec((num_graphs, out_dim), const),
        out_shape=jax.ShapeDtypeStruct((num_graphs, out_dim), jnp.float32),
        scratch_shapes=[
            pltpu.VMEM((_NB, 1), jnp.int32),
            pltpu.VMEM((num_graphs, hidden + 1), jnp.float32),
        ],
        compiler_params=pltpu.CompilerParams(
            dimension_semantics=("arbitrary",)),
    )(src, attr, b32, W1.astype(jnp.bfloat16), b1.reshape(1, hidden), W2,
      b2.reshape(1, hidden), W3, b3.reshape(1, out_dim))
    return out


# speedup vs baseline: 1.4623x; 1.0037x over previous
import functools

import jax
import jax.numpy as jnp
from jax.experimental import pallas as pl
from jax.experimental.pallas import tpu as pltpu

_EDGE_TILE = 32000
_NODE_CHUNK = 800
_NB = 136


def _fused_kernel(num_tiles, num_graphs, node_rows,
                  src_ref, attr_ref, batch_ref, w1_ref, b1_ref, w2_ref,
                  b2_ref, w3_ref, b3_ref, out_ref, starts_ref, acc_ref):
    i = pl.program_id(0)
    et = attr_ref.shape[0]
    hidden = w2_ref.shape[0]

    @pl.when(i == 0)
    def _init():
        gcol = jax.lax.broadcasted_iota(jnp.int32, (_NB, 1), 0)

        def body(j, s):
            row = batch_ref[pl.ds(j, 1), :]
            return s + jnp.sum((gcol > row).astype(jnp.int32), axis=1,
                               keepdims=True)

        zero = jnp.zeros((_NB, 1), jnp.int32)
        starts_ref[...] = jax.lax.fori_loop(0, node_rows, body, zero)
        acc_ref[...] = jnp.zeros_like(acc_ref)

    node = src_ref[0]
    one = jnp.ones((), jnp.float32)
    zro = jnp.zeros((), jnp.float32)
    u = jnp.where(node >= starts_ref[...], one, zro).astype(jnp.bfloat16)
    h = jnp.dot(attr_ref[...], w1_ref[...],
                preferred_element_type=jnp.float32)
    h = jnp.maximum(h + b1_ref[...], 0.0)
    h_aug = jnp.concatenate(
        [h.astype(jnp.bfloat16), jnp.ones((et, 1), jnp.bfloat16)], axis=1)
    p = jax.lax.dot_general(u, h_aug, (((1,), (0,)), ((), ())),
                            preferred_element_type=jnp.float32)
    acc_ref[...] += p[0:num_graphs, :] - p[1:num_graphs + 1, :]

    @pl.when(i == num_tiles - 1)
    def _finish():
        a = acc_ref[...]
        gf = a[:, 0:hidden] / jnp.maximum(a[:, hidden:hidden + 1], 1.0)
        h2 = jnp.dot(gf, w2_ref[...], preferred_element_type=jnp.float32)
        h2 = jnp.maximum(h2 + b2_ref[...], 0.0)
        out = jnp.dot(h2, w3_ref[...], preferred_element_type=jnp.float32)
        out_ref[...] = out + b3_ref[...]


@jax.jit
def kernel(edge_index, edge_attr, batch, W1, b1, W2, b2, W3, b3):
    n_edges, in_dim = edge_attr.shape
    n_nodes = batch.shape[0]
    hidden = W1.shape[1]
    out_dim = W3.shape[1]
    num_graphs = 128

    et = _EDGE_TILE
    pad_e = (-n_edges) % et
    src = edge_index[0].astype(jnp.int32)
    attr = edge_attr.astype(jnp.bfloat16)
    if pad_e:
        src = jnp.concatenate(
            [src, jnp.full((pad_e,), n_nodes, jnp.int32)])
        attr = jnp.concatenate(
            [attr, jnp.zeros((pad_e, in_dim), attr.dtype)])
    num_tiles = (n_edges + pad_e) // et
    src = src.reshape(num_tiles, 1, et)

    b32 = batch.astype(jnp.int32)
    pad_n = (-n_nodes) % _NODE_CHUNK
    if pad_n:
        b32 = jnp.concatenate(
            [b32, jnp.full((pad_n,), num_graphs, jnp.int32)])
    node_rows = (n_nodes + pad_n) // _NODE_CHUNK
    b32 = b32.reshape(node_rows, _NODE_CHUNK)

    const = lambda i: (0, 0)
    out = pl.pallas_call(
        functools.partial(_fused_kernel, num_tiles, num_graphs, node_rows),
        grid=(num_tiles,),
        in_specs=[
            pl.BlockSpec((1, 1, et), lambda i: (i, 0, 0)),
            pl.BlockSpec((et, in_dim), lambda i: (i, 0)),
            pl.BlockSpec((node_rows, _NODE_CHUNK), const),
            pl.BlockSpec((in_dim, hidden), const),
            pl.BlockSpec((1, hidden), const),
            pl.BlockSpec((hidden, hidden), const),
            pl.BlockSpec((1, hidden), const),
            pl.BlockSpec((hidden, out_dim), const),
            pl.BlockSpec((1, out_dim), const),
        ],
        out_specs=pl.BlockSpec((num_graphs, out_dim), const),
        out_shape=jax.ShapeDtypeStruct((num_graphs, out_dim), jnp.float32),
        scratch_shapes=[
            pltpu.VMEM((_NB, 1), jnp.int32),
            pltpu.VMEM((num_graphs, hidden + 1), jnp.float32),
        ],
        compiler_params=pltpu.CompilerParams(
            dimension_semantics=("arbitrary",)),
    )(src, attr, b32, W1.astype(jnp.bfloat16), b1.reshape(1, hidden), W2,
      b2.reshape(1, hidden), W3, b3.reshape(1, out_dim))
    return out
